# baseline (device time: 93912 ns/iter reference)
import jax
import jax.numpy as jnp
from jax import lax
from jax.experimental import pallas as pl
from jax.experimental.pallas import tpu as pltpu

B = 8
H = 8
D = 128
BS = 16
P_LOCAL = 512
P_X = P_LOCAL // 2
NSLOTS = 512
CP = 64
CKL = CP * BS
N_CHUNKS = P_X // CP
NEG_INF = -1e30


def kernel(Q, K, V, bt, lens):
    xy = jnp.stack([lax.axis_index("x"), lax.axis_index("y")])

    q_t = (Q[:, 0, :, :] * (D ** -0.5)).transpose(1, 0, 2)
    eye = jnp.eye(H, dtype=q_t.dtype)
    q_blk = (q_t[:, :, None, :] * eye[:, None, :, None]).reshape(H * B, H * D)
    q_blk = q_blk.astype(jnp.bfloat16)

    K2 = K.reshape(P_LOCAL, BS, H * D)
    V2 = V.reshape(P_LOCAL, BS, H * D)

    slot = jnp.arange(NSLOTS, dtype=jnp.int32)[None, :]
    btT = jnp.where(slot < lens[:, None], bt, -1).T

    def body(xy_ref, q_ref, k_ref, v_ref, btT_ref, out_ref,
             acc_s, m_s, l_s, acc_buf, stat_buf, send_sems, recv_sems):
        c = pl.program_id(0)
        my_x = xy_ref[0]
        my_y = xy_ref[1]
        peers = [(1 - my_x, my_y), (my_x, 1 - my_y), (1 - my_x, 1 - my_y)]

        @pl.when(c == 0)
        def _init():
            acc_s[...] = jnp.zeros((H, B, D), jnp.float32)
            m_s[...] = jnp.full((H, B), NEG_INF, jnp.float32)
            l_s[...] = jnp.zeros((H, B), jnp.float32)

        base = my_y * P_LOCAL + my_x * P_X + c * CP
        counts_rows = []
        for b in range(B):
            bt_col = btT_ref[:, b:b + 1]
            pg = lax.broadcasted_iota(jnp.int32, (NSLOTS, CP), 1) + base
            match = (bt_col == pg).astype(jnp.float32)
            counts_rows.append(jnp.sum(match, axis=0, keepdims=True))
        counts_pg = jnp.concatenate(counts_rows, axis=0)

        row = lax.broadcasted_iota(jnp.int32, (CP, CKL), 0)
        col = lax.broadcasted_iota(jnp.int32, (CP, CKL), 1)
        expand = (col // BS == row).astype(jnp.float32)
        counts = lax.dot_general(
            counts_pg, expand,
            dimension_numbers=(((1,), (0,)), ((), ())),
            preferred_element_type=jnp.float32,
        )

        k2 = k_ref[...].reshape(CKL, H * D).astype(jnp.bfloat16)
        v2 = v_ref[...].reshape(CKL, H * D).astype(jnp.bfloat16)

        s_big = lax.dot_general(
            q_ref[...], k2,
            dimension_numbers=(((1,), (1,)), ((), ())),
            preferred_element_type=jnp.float32,
        )
        s = s_big.reshape(H, B, CKL)

        m_old = m_s[...]
        cm = jnp.max(s, axis=2)
        m_new = jnp.maximum(m_old, cm)
        alpha = jnp.exp(m_old - m_new)
        p = jnp.exp(s - m_new[:, :, None]) * counts[None]
        l_s[...] = l_s[...] * alpha + jnp.sum(p, axis=2)

        p_big = p.reshape(H * B, CKL).astype(jnp.bfloat16)
        o_big = lax.dot_general(
            p_big, v2,
            dimension_numbers=(((1,), (0,)), ((), ())),
            preferred_element_type=jnp.float32,
        )
        o3 = o_big.reshape(H, B, H * D)
        pv = jnp.concatenate(
            [o3[h:h + 1, :, h * D:(h + 1) * D] for h in range(H)], axis=0
        )
        acc_s[...] = acc_s[...] * alpha[:, :, None] + pv
        m_s[...] = m_new

        @pl.when(c == N_CHUNKS - 1)
        def _exchange_and_combine():
            barrier = pltpu.get_barrier_semaphore()
            for nbr in peers:
                pl.semaphore_signal(barrier, inc=1, device_id=nbr,
                                    device_id_type=pl.DeviceIdType.MESH)
            pl.semaphore_wait(barrier, 3)

            acc_buf[0] = acc_s[...]
            stat_buf[0, 0] = m_s[...]
            stat_buf[0, 1] = l_s[...]

            rdmas = []
            for i, nbr in enumerate(peers):
                rdmas.append(pltpu.make_async_remote_copy(
                    src_ref=acc_buf.at[0], dst_ref=acc_buf.at[1 + i],
                    send_sem=send_sems.at[2 * i], recv_sem=recv_sems.at[2 * i],
                    device_id=nbr, device_id_type=pl.DeviceIdType.MESH,
                ))
                rdmas.append(pltpu.make_async_remote_copy(
                    src_ref=stat_buf.at[0], dst_ref=stat_buf.at[1 + i],
                    send_sem=send_sems.at[2 * i + 1],
                    recv_sem=recv_sems.at[2 * i + 1],
                    device_id=nbr, device_id_type=pl.DeviceIdType.MESH,
                ))
            for r in rdmas:
                r.start()
            for r in rdmas:
                r.wait()

            ms = [stat_buf[i, 0] for i in range(4)]
            ls = [stat_buf[i, 1] for i in range(4)]
            m_g = jnp.maximum(jnp.maximum(ms[0], ms[1]),
                              jnp.maximum(ms[2], ms[3]))
            l_g = jnp.zeros((H, B), jnp.float32)
            acc_g = jnp.zeros((H, B, D), jnp.float32)
            for i in range(4):
                w = jnp.exp(ms[i] - m_g)
                l_g = l_g + ls[i] * w
                acc_g = acc_g + acc_buf[i] * w[:, :, None]

            o = acc_g / l_g[:, :, None]
            out_ref[...] = o.transpose(1, 0, 2).reshape(B, 1, H, D)

    grid_spec = pltpu.PrefetchScalarGridSpec(
        num_scalar_prefetch=1,
        grid=(N_CHUNKS,),
        in_specs=[
            pl.BlockSpec((H * B, H * D), lambda c, xy: (0, 0)),
            pl.BlockSpec((CP, BS, H * D),
                         lambda c, xy: (xy[0] * N_CHUNKS + c, 0, 0)),
            pl.BlockSpec((CP, BS, H * D),
                         lambda c, xy: (xy[0] * N_CHUNKS + c, 0, 0)),
            pl.BlockSpec((NSLOTS, B), lambda c, xy: (0, 0)),
        ],
        out_specs=pl.BlockSpec((B, 1, H, D), lambda c, xy: (0, 0, 0, 0)),
        scratch_shapes=[
            pltpu.VMEM((H, B, D), jnp.float32),
            pltpu.VMEM((H, B), jnp.float32),
            pltpu.VMEM((H, B), jnp.float32),
            pltpu.VMEM((4, H, B, D), jnp.float32),
            pltpu.VMEM((4, 2, H, B), jnp.float32),
            pltpu.SemaphoreType.DMA((6,)),
            pltpu.SemaphoreType.DMA((6,)),
        ],
    )
    return pl.pallas_call(
        body,
        grid_spec=grid_spec,
        out_shape=jax.ShapeDtypeStruct((B, 1, H, D), jnp.float32),
        compiler_params=pltpu.CompilerParams(
            collective_id=0,
            dimension_semantics=("arbitrary",),
        ),
    )(xy, q_blk, K2, V2, btT)


# device time: 32718 ns/iter; 2.8703x vs baseline; 2.8703x over previous
import jax
import jax.numpy as jnp
from jax import lax
from jax.experimental import pallas as pl
from jax.experimental.pallas import tpu as pltpu

B = 8
H = 8
D = 128
BS = 16
P_LOCAL = 512
P_X = P_LOCAL // 2
NSLOTS = 512
CP = 64
CKL = CP * BS
N_CHUNKS = P_X // CP
NEG_INF = -1e30

DISABLE_HISTOGRAM = False


def kernel(Q, K, V, bt, lens):
    xy = jnp.stack([lax.axis_index("x"), lax.axis_index("y")])

    def body(xy_ref, lens_ref, q_ref, k_ref, v_ref, bt_ref, out_ref,
             acc_s, m_s, l_s, acc_buf, stat_buf, send_sems, recv_sems):
        c = pl.program_id(0)
        my_x = xy_ref[0]
        my_y = xy_ref[1]
        peer_x = (1 - my_x, my_y)
        peer_y = (my_x, 1 - my_y)

        @pl.when(c == 0)
        def _init():
            acc_s[...] = jnp.zeros((H, B, D), jnp.float32)
            m_s[...] = jnp.full((H, B), NEG_INF, jnp.float32)
            l_s[...] = jnp.zeros((H, B), jnp.float32)

        base = my_y * P_LOCAL + my_x * P_X + c * CP
        if DISABLE_HISTOGRAM:
            counts = jnp.ones((B, CKL), jnp.float32)
        else:
            counts_rows = []
            for b in range(B):
                bt_row = bt_ref[b:b + 1, :]
                pg = lax.broadcasted_iota(jnp.int32, (CP, NSLOTS), 0) + base
                sl = lax.broadcasted_iota(jnp.int32, (CP, NSLOTS), 1)
                len_b = lens_ref[b]
                match = (bt_row == pg) & (sl < len_b)
                cnt = jnp.sum(match.astype(jnp.float32), axis=1, keepdims=True)
                counts_rows.append(cnt.T)
            counts_pg = jnp.concatenate(counts_rows, axis=0)

            row = lax.broadcasted_iota(jnp.int32, (CP, CKL), 0)
            col = lax.broadcasted_iota(jnp.int32, (CP, CKL), 1)
            expand = (col // BS == row).astype(jnp.float32)
            counts = lax.dot_general(
                counts_pg, expand,
                dimension_numbers=(((1,), (0,)), ((), ())),
                preferred_element_type=jnp.float32,
            )

        q = q_ref[:, 0, :, :]
        k = k_ref[...].reshape(CKL, H, D)
        v = v_ref[...].reshape(CKL, H, D)

        s_heads = []
        for h in range(H):
            s_h = lax.dot_general(
                q[:, h, :], k[:, h, :],
                dimension_numbers=(((1,), (1,)), ((), ())),
                preferred_element_type=jnp.float32,
            )
            s_heads.append(s_h[None])
        s = jnp.concatenate(s_heads, axis=0) * (D ** -0.5)

        m_old = m_s[...]
        cm = jnp.max(s, axis=2)
        m_new = jnp.maximum(m_old, cm)
        alpha = jnp.exp(m_old - m_new)
        p = jnp.exp(s - m_new[:, :, None]) * counts[None]
        l_s[...] = l_s[...] * alpha + jnp.sum(p, axis=2)
        pv_heads = []
        for h in range(H):
            pv_h = lax.dot_general(
                p[h], v[:, h, :],
                dimension_numbers=(((1,), (0,)), ((), ())),
                preferred_element_type=jnp.float32,
            )
            pv_heads.append(pv_h[None])
        pv = jnp.concatenate(pv_heads, axis=0)
        acc_s[...] = acc_s[...] * alpha[:, :, None] + pv
        m_s[...] = m_new

        @pl.when(c == N_CHUNKS - 1)
        def _exchange_and_combine():
            barrier = pltpu.get_barrier_semaphore()
            for nbr in (peer_x, peer_y):
                pl.semaphore_signal(barrier, inc=1, device_id=nbr,
                                    device_id_type=pl.DeviceIdType.MESH)
            pl.semaphore_wait(barrier, 2)

            def exchange(peer, recv_slot, sem_base):
                rdma_acc = pltpu.make_async_remote_copy(
                    src_ref=acc_buf.at[0], dst_ref=acc_buf.at[recv_slot],
                    send_sem=send_sems.at[sem_base], recv_sem=recv_sems.at[sem_base],
                    device_id=peer, device_id_type=pl.DeviceIdType.MESH,
                )
                rdma_stat = pltpu.make_async_remote_copy(
                    src_ref=stat_buf.at[0], dst_ref=stat_buf.at[recv_slot],
                    send_sem=send_sems.at[sem_base + 1],
                    recv_sem=recv_sems.at[sem_base + 1],
                    device_id=peer, device_id_type=pl.DeviceIdType.MESH,
                )
                rdma_acc.start()
                rdma_stat.start()
                rdma_acc.wait()
                rdma_stat.wait()

            def combine(slot):
                m_a = stat_buf[0, 0]
                l_a = stat_buf[0, 1]
                m_b = stat_buf[slot, 0]
                l_b = stat_buf[slot, 1]
                m_g = jnp.maximum(m_a, m_b)
                w_a = jnp.exp(m_a - m_g)
                w_b = jnp.exp(m_b - m_g)
                l_g = l_a * w_a + l_b * w_b
                acc_g = (acc_buf[0] * w_a[:, :, None]
                         + acc_buf[slot] * w_b[:, :, None])
                return m_g, l_g, acc_g

            acc_buf[0] = acc_s[...]
            stat_buf[0, 0] = m_s[...]
            stat_buf[0, 1] = l_s[...]

            exchange(peer_x, 1, 0)
            m_g, l_g, acc_g = combine(1)
            acc_buf[0] = acc_g
            stat_buf[0, 0] = m_g
            stat_buf[0, 1] = l_g

            exchange(peer_y, 2, 2)
            m_g, l_g, acc_g = combine(2)

            o = acc_g / l_g[:, :, None]
            out_ref[...] = o.transpose(1, 0, 2).reshape(B, 1, H, D)

    grid_spec = pltpu.PrefetchScalarGridSpec(
        num_scalar_prefetch=2,
        grid=(N_CHUNKS,),
        in_specs=[
            pl.BlockSpec((B, 1, H, D), lambda c, xy, lens: (0, 0, 0, 0)),
            pl.BlockSpec((CP, BS, H, D),
                         lambda c, xy, lens: (xy[0] * N_CHUNKS + c, 0, 0, 0)),
            pl.BlockSpec((CP, BS, H, D),
                         lambda c, xy, lens: (xy[0] * N_CHUNKS + c, 0, 0, 0)),
            pl.BlockSpec((B, NSLOTS), lambda c, xy, lens: (0, 0)),
        ],
        out_specs=pl.BlockSpec((B, 1, H, D), lambda c, xy, lens: (0, 0, 0, 0)),
        scratch_shapes=[
            pltpu.VMEM((H, B, D), jnp.float32),
            pltpu.VMEM((H, B), jnp.float32),
            pltpu.VMEM((H, B), jnp.float32),
            pltpu.VMEM((3, H, B, D), jnp.float32),
            pltpu.VMEM((3, 2, H, B), jnp.float32),
            pltpu.SemaphoreType.DMA((4,)),
            pltpu.SemaphoreType.DMA((4,)),
        ],
    )
    return pl.pallas_call(
        body,
        grid_spec=grid_spec,
        out_shape=jax.ShapeDtypeStruct((B, 1, H, D), jnp.float32),
        compiler_params=pltpu.CompilerParams(
            collective_id=0,
            dimension_semantics=("arbitrary",),
        ),
    )(xy, lens, Q, K, V, bt)
